# baseline (speedup 1.0000x reference)
import functools

import jax
import jax.numpy as jnp
from jax import lax
from jax.experimental import pallas as pl
from jax.experimental.pallas import tpu as pltpu, tpu_sc as plsc

N_VOCAB_ = 100000
D = 1024
DW = D // 2

_info = plsc.get_sparse_core_info()
NC, NS, L = _info.num_cores, _info.num_subcores, _info.num_lanes
NW = NC * NS

B = 4 * 2048
B_PER_W = B // NW
CHUNK = 32
N_CHUNKS = B_PER_W // CHUNK

_mesh = plsc.VectorSubcoreMesh(core_axis_name="c", subcore_axis_name="s")


@functools.partial(
    pl.kernel,
    mesh=_mesh,
    out_type=jax.ShapeDtypeStruct((NW, N_CHUNKS, CHUNK, DW), jnp.int32),
    scratch_types=[
        pltpu.VMEM((N_CHUNKS, CHUNK), jnp.int32),
        pltpu.VMEM((CHUNK, D), jnp.float32),
        pltpu.VMEM((CHUNK, D), jnp.float32),
        pltpu.VMEM((CHUNK, DW), jnp.int32),
        pltpu.VMEM((CHUNK, DW), jnp.int32),
        pltpu.SemaphoreType.DMA,
        pltpu.SemaphoreType.DMA,
    ],
    compiler_params=pltpu.CompilerParams(
        use_tc_tiling_on_sc=False, needs_layout_passes=False),
)
def _embed_sc(idx_hbm, table_hbm, out_hbm, idx_v, rows0, rows1, outb0, outb1,
              gsem, osem):
    wid = lax.axis_index("s") * NC + lax.axis_index("c")
    pltpu.sync_copy(idx_hbm.at[wid], idx_v)

    iota = lax.iota(jnp.int32, L)
    ev_lane = iota * 2

    rows_bufs = (rows0, rows1)
    out_bufs = (outb0, outb1)

    def convert_chunk(rows_ref, out_ref):
        def body(k, _):
            r = k >> 5
            c32 = (k & 31) * 32
            rvec = lax.broadcast(r, (L,))
            a = plsc.load_gather(rows_ref, [rvec, c32 + ev_lane])
            b = plsc.load_gather(rows_ref, [rvec, c32 + ev_lane + 1])
            ua = plsc.bitcast(a, jnp.int32)
            ub = plsc.bitcast(b, jnp.int32)
            ta = ua + 0x7FFF + ((ua >> 16) & 1)
            tb = ub + 0x7FFF + ((ub >> 16) & 1)
            word = lax.shift_right_logical(ta, 16) | (tb & jnp.int32(-0x10000))
            out_ref[r, pl.ds((k & 31) * 16, L)] = word
            return 0

        lax.fori_loop(0, CHUNK * 32, body, 0)

    gh = [None] * N_CHUNKS
    oh = [None] * N_CHUNKS
    gh[0] = pltpu.async_copy(table_hbm.at[idx_v.at[0]], rows0, gsem)
    for g in range(N_CHUNKS):
        gh[g].wait()
        if g + 1 < N_CHUNKS:
            gh[g + 1] = pltpu.async_copy(
                table_hbm.at[idx_v.at[g + 1]], rows_bufs[(g + 1) % 2], gsem)
        if g >= 2:
            oh[g - 2].wait()
        convert_chunk(rows_bufs[g % 2], out_bufs[g % 2])
        oh[g] = pltpu.async_copy(out_bufs[g % 2], out_hbm.at[wid, g], osem)
    oh[N_CHUNKS - 2].wait()
    oh[N_CHUNKS - 1].wait()


def kernel(xBT, embedding):
    idx = xBT.reshape(NW, N_CHUNKS, CHUNK)
    packed = _embed_sc(idx, embedding)
    out = lax.bitcast_convert_type(packed, jnp.bfloat16)
    return out.reshape(4, 2048, D)


# speedup vs baseline: 1.0032x; 1.0032x over previous
import functools

import jax
import jax.numpy as jnp
from jax import lax
from jax.experimental import pallas as pl
from jax.experimental.pallas import tpu as pltpu, tpu_sc as plsc

N_VOCAB_ = 100000
D = 1024
DW = D // 2

_info = plsc.get_sparse_core_info()
NC, NS, L = _info.num_cores, _info.num_subcores, _info.num_lanes
NW = NC * NS

B = 4 * 2048
B_PER_W = B // NW
CHUNK = 32
N_CHUNKS = B_PER_W // CHUNK

_mesh = plsc.VectorSubcoreMesh(core_axis_name="c", subcore_axis_name="s")


@functools.partial(
    pl.kernel,
    mesh=_mesh,
    out_type=jax.ShapeDtypeStruct((NW, N_CHUNKS, CHUNK, DW), jnp.int32),
    scratch_types=[
        pltpu.VMEM((N_CHUNKS, CHUNK), jnp.int32),
        pltpu.VMEM((CHUNK, D), jnp.float32),
        pltpu.VMEM((CHUNK, D), jnp.float32),
        pltpu.VMEM((CHUNK, DW), jnp.int32),
        pltpu.VMEM((CHUNK, DW), jnp.int32),
        pltpu.SemaphoreType.DMA,
        pltpu.SemaphoreType.DMA,
    ],
    compiler_params=pltpu.CompilerParams(
        use_tc_tiling_on_sc=False, needs_layout_passes=False),
)
def _embed_sc(idx_hbm, table_hbm, out_hbm, idx_v, rows0, rows1, outb0, outb1,
              gsem, osem):
    wid = lax.axis_index("s") * NC + lax.axis_index("c")
    pltpu.sync_copy(idx_hbm.at[wid], idx_v)

    iota = lax.iota(jnp.int32, L)
    ev_lane = iota * 2

    rows_bufs = (rows0, rows1)
    out_bufs = (outb0, outb1)

    def convert_chunk(rows_ref, out_ref):
        def row_body(r, _):
            rvec = lax.broadcast(r, (L,))
            for j in range(32):
                a = plsc.load_gather(rows_ref, [rvec, j * 32 + ev_lane])
                b = plsc.load_gather(rows_ref, [rvec, j * 32 + ev_lane + 1])
                ua = plsc.bitcast(a, jnp.int32)
                ub = plsc.bitcast(b, jnp.int32)
                ta = ua + 0x7FFF + ((ua >> 16) & 1)
                tb = ub + 0x7FFF + ((ub >> 16) & 1)
                word = (lax.shift_right_logical(ta, 16)
                        | (tb & jnp.int32(-0x10000)))
                out_ref[r, pl.ds(j * 16, L)] = word
            return 0

        lax.fori_loop(0, CHUNK, row_body, 0)

    gh = [None] * N_CHUNKS
    oh = [None] * N_CHUNKS
    gh[0] = pltpu.async_copy(table_hbm.at[idx_v.at[0]], rows0, gsem)
    for g in range(N_CHUNKS):
        gh[g].wait()
        if g + 1 < N_CHUNKS:
            gh[g + 1] = pltpu.async_copy(
                table_hbm.at[idx_v.at[g + 1]], rows_bufs[(g + 1) % 2], gsem)
        if g >= 2:
            oh[g - 2].wait()
        convert_chunk(rows_bufs[g % 2], out_bufs[g % 2])
        oh[g] = pltpu.async_copy(out_bufs[g % 2], out_hbm.at[wid, g], osem)
    oh[N_CHUNKS - 2].wait()
    oh[N_CHUNKS - 1].wait()


def kernel(xBT, embedding):
    idx = xBT.reshape(NW, N_CHUNKS, CHUNK)
    packed = _embed_sc(idx, embedding)
    out = lax.bitcast_convert_type(packed, jnp.bfloat16)
    return out.reshape(4, 2048, D)
